# baseline (speedup 1.0000x reference)
import dataclasses
import functools

import jax
import jax.numpy as jnp
from jax import lax
from jax.experimental import pallas as pl
from jax.experimental.pallas import tpu as pltpu
from jax.experimental.pallas import tpu_sc as plsc

NUM_CORES = 2
NUM_SUBCORES = 16
NUM_WORKERS = NUM_CORES * NUM_SUBCORES
LANES = 16
FACTORS = 64


def _build(batch):
    chunk = batch // NUM_WORKERS
    mesh = plsc.VectorSubcoreMesh(core_axis_name="c", subcore_axis_name="s")
    cp = pltpu.CompilerParams(
        needs_layout_passes=False, use_tc_tiling_on_sc=False)

    @functools.partial(
        pl.kernel,
        out_type=jax.ShapeDtypeStruct((batch,), jnp.float32),
        mesh=mesh,
        compiler_params=cp,
        scratch_types=[
            pltpu.VMEM((chunk,), jnp.int32),
            pltpu.VMEM((chunk,), jnp.int32),
            pltpu.VMEM((chunk, FACTORS), jnp.float32),
            pltpu.VMEM((chunk, FACTORS), jnp.float32),
            pltpu.VMEM((chunk,), jnp.float32),
            pltpu.VMEM((chunk,), jnp.float32),
            pltpu.VMEM((LANES,), jnp.float32),
            pltpu.VMEM((chunk,), jnp.float32),
            pltpu.SemaphoreType.DMA,
            pltpu.SemaphoreType.DMA,
            pltpu.SemaphoreType.DMA,
            pltpu.SemaphoreType.DMA,
        ],
    )
    def svd_kernel(uidx_hbm, midx_hbm, utab_hbm, mtab_hbm, ub_hbm, mb_hbm,
                   gb_hbm, out_hbm, uidx_v, midx_v, urows_v, mrows_v,
                   ub_v, mb_v, gb_v, out_v, sem0, sem1, sem2, sem3):
        wid = lax.axis_index("s") * NUM_CORES + lax.axis_index("c")
        base = wid * chunk

        pltpu.sync_copy(uidx_hbm.at[pl.ds(base, chunk)], uidx_v)
        pltpu.sync_copy(midx_hbm.at[pl.ds(base, chunk)], midx_v)
        pltpu.sync_copy(gb_hbm, gb_v)

        cp0 = pltpu.async_copy(utab_hbm.at[uidx_v], urows_v, sem0)
        cp1 = pltpu.async_copy(mtab_hbm.at[midx_v], mrows_v, sem1)
        cp2 = pltpu.async_copy(ub_hbm.at[uidx_v], ub_v, sem2)
        cp3 = pltpu.async_copy(mb_hbm.at[midx_v], mb_v, sem3)
        cp2.wait()
        cp3.wait()
        cp0.wait()
        cp1.wait()

        gb = gb_v[...]
        iota = lax.broadcasted_iota(jnp.int32, (LANES,), 0)

        @pl.loop(0, chunk, step=LANES)
        def _(g):
            rows = g + iota
            acc = ub_v[pl.ds(g, LANES)] + mb_v[pl.ds(g, LANES)] + gb
            for f in range(FACTORS):
                col = jnp.full((LANES,), f, jnp.int32)
                uv = plsc.load_gather(urows_v, [rows, col])
                mv = plsc.load_gather(mrows_v, [rows, col])
                acc = acc + uv * mv
            out_v[pl.ds(g, LANES)] = acc

        pltpu.sync_copy(out_v, out_hbm.at[pl.ds(base, chunk)])

    return svd_kernel


def kernel(user_indices, movie_indices, user_embedding, movie_embedding,
           user_bias, movie_bias, global_bias):
    batch = user_indices.shape[0]
    k = _build(batch)
    return k(
        user_indices.astype(jnp.int32),
        movie_indices.astype(jnp.int32),
        user_embedding,
        movie_embedding,
        jnp.reshape(user_bias, (-1,)),
        jnp.reshape(movie_bias, (-1,)),
        jnp.broadcast_to(global_bias, (LANES,)).astype(jnp.float32),
    )
